# baseline (speedup 1.0000x reference)
import jax
import jax.numpy as jnp
from jax.experimental import pallas as pl
from jax.experimental.pallas import tpu as pltpu

NA = 3
NO = 85
NK = 2


def _detect_kernel(x0_ref, x1_ref, x2_ref, w0_ref, w1_ref, w2_ref, b_ref,
                   o0_ref, o1_ref, o2_ref, acc0, acc1, acc2):
    k = pl.program_id(1)
    for lvl, (x_ref, w_ref, o_ref, acc) in enumerate(
            ((x0_ref, w0_ref, o0_ref, acc0),
             (x1_ref, w1_ref, o1_ref, acc1),
             (x2_ref, w2_ref, o2_ref, acc2))):
        partial = jax.lax.dot_general(
            x_ref[0], w_ref[...],
            dimension_numbers=(((0,), (0,)), ((), ())),
            preferred_element_type=jnp.float32,
        )

        @pl.when(k == 0)
        def _():
            acc[...] = partial

        @pl.when(k == NK - 1)
        def _():
            res = acc[...] + partial + b_ref[lvl]
            for a in range(NA):
                o_ref[0, a] = res[:, a * NO:(a + 1) * NO]


def kernel(x0, x1, x2, W0, b0, W1, b1, W2, b2):
    bs = x0.shape[0]
    shapes = [x.shape for x in (x0, x1, x2)]
    cs = [c for (_, c, _, _) in shapes]
    hws = [ny * nx for (_, _, ny, nx) in shapes]
    xrs = [x.reshape(x.shape[0], x.shape[1], -1) for x in (x0, x1, x2)]
    wts = [W.T for W in (W0, W1, W2)]
    br = jnp.stack([b0, b1, b2]).reshape(3, 1, NA * NO)

    outs = pl.pallas_call(
        _detect_kernel,
        grid=(bs, NK),
        in_specs=[
            pl.BlockSpec((1, cs[0] // NK, hws[0]), lambda g, k: (g, k, 0)),
            pl.BlockSpec((1, cs[1] // NK, hws[1]), lambda g, k: (g, k, 0)),
            pl.BlockSpec((1, cs[2] // NK, hws[2]), lambda g, k: (g, k, 0)),
            pl.BlockSpec((cs[0] // NK, NA * NO), lambda g, k: (k, 0)),
            pl.BlockSpec((cs[1] // NK, NA * NO), lambda g, k: (k, 0)),
            pl.BlockSpec((cs[2] // NK, NA * NO), lambda g, k: (k, 0)),
            pl.BlockSpec((3, 1, NA * NO), lambda g, k: (0, 0, 0)),
        ],
        out_specs=[
            pl.BlockSpec((1, NA, hws[0], NO), lambda g, k: (g, 0, 0, 0)),
            pl.BlockSpec((1, NA, hws[1], NO), lambda g, k: (g, 0, 0, 0)),
            pl.BlockSpec((1, NA, hws[2], NO), lambda g, k: (g, 0, 0, 0)),
        ],
        out_shape=[
            jax.ShapeDtypeStruct((bs, NA, hws[0], NO), jnp.float32),
            jax.ShapeDtypeStruct((bs, NA, hws[1], NO), jnp.float32),
            jax.ShapeDtypeStruct((bs, NA, hws[2], NO), jnp.float32),
        ],
        scratch_shapes=[
            pltpu.VMEM((hws[0], NA * NO), jnp.float32),
            pltpu.VMEM((hws[1], NA * NO), jnp.float32),
            pltpu.VMEM((hws[2], NA * NO), jnp.float32),
        ],
        compiler_params=pltpu.CompilerParams(
            dimension_semantics=("parallel", "arbitrary")),
    )(xrs[0], xrs[1], xrs[2], wts[0], wts[1], wts[2], br)
    return tuple(
        o.reshape(bs, NA, ny, nx, NO)
        for o, (_, _, ny, nx) in zip(outs, shapes))


# speedup vs baseline: 1.3203x; 1.3203x over previous
import jax
import jax.numpy as jnp
from jax.experimental import pallas as pl
from jax.experimental.pallas import tpu as pltpu

NA = 3
NO = 85


def _detect_kernel(x0_ref, x1_ref, x2_ref, w0_ref, w1_ref, w2_ref,
                   b_ref, o0_ref, o1_ref, o2_ref):
    for lvl, (x_ref, w_ref, o_ref) in enumerate(
            ((x0_ref, w0_ref, o0_ref),
             (x1_ref, w1_ref, o1_ref),
             (x2_ref, w2_ref, o2_ref))):
        res = jax.lax.dot_general(
            x_ref[0], w_ref[...],
            dimension_numbers=(((0,), (0,)), ((), ())),
            preferred_element_type=jnp.float32,
        )
        res = res + b_ref[lvl]
        for a in range(NA):
            o_ref[0, a] = res[:, a * NO:(a + 1) * NO]


def kernel(x0, x1, x2, W0, b0, W1, b1, W2, b2):
    bs = x0.shape[0]
    shapes = [x.shape for x in (x0, x1, x2)]
    hws = [ny * nx for (_, _, ny, nx) in shapes]
    xrs = [x.reshape(x.shape[0], x.shape[1], -1) for x in (x0, x1, x2)]
    wts = [W.T for W in (W0, W1, W2)]
    br = jnp.stack([b0, b1, b2]).reshape(3, 1, NA * NO)

    outs = pl.pallas_call(
        _detect_kernel,
        grid=(bs,),
        in_specs=[
            pl.BlockSpec((1, shapes[0][1], hws[0]), lambda g: (g, 0, 0)),
            pl.BlockSpec((1, shapes[1][1], hws[1]), lambda g: (g, 0, 0)),
            pl.BlockSpec((1, shapes[2][1], hws[2]), lambda g: (g, 0, 0)),
            pl.BlockSpec((shapes[0][1], NA * NO), lambda g: (0, 0)),
            pl.BlockSpec((shapes[1][1], NA * NO), lambda g: (0, 0)),
            pl.BlockSpec((shapes[2][1], NA * NO), lambda g: (0, 0)),
            pl.BlockSpec((3, 1, NA * NO), lambda g: (0, 0, 0)),
        ],
        out_specs=[
            pl.BlockSpec((1, NA, hws[0], NO), lambda g: (g, 0, 0, 0)),
            pl.BlockSpec((1, NA, hws[1], NO), lambda g: (g, 0, 0, 0)),
            pl.BlockSpec((1, NA, hws[2], NO), lambda g: (g, 0, 0, 0)),
        ],
        out_shape=[
            jax.ShapeDtypeStruct((bs, NA, hws[0], NO), jnp.float32),
            jax.ShapeDtypeStruct((bs, NA, hws[1], NO), jnp.float32),
            jax.ShapeDtypeStruct((bs, NA, hws[2], NO), jnp.float32),
        ],
        compiler_params=pltpu.CompilerParams(
            dimension_semantics=("parallel",)),
    )(xrs[0], xrs[1], xrs[2], wts[0], wts[1], wts[2], br)
    return tuple(
        o.reshape(bs, NA, ny, nx, NO)
        for o, (_, _, ny, nx) in zip(outs, shapes))
